# baseline (speedup 1.0000x reference)
import functools

import jax
import jax.numpy as jnp
from jax import lax
from jax.experimental import pallas as pl
from jax.experimental.pallas import tpu as pltpu
from jax.experimental.pallas import tpu_sc as plsc

NC = 2
NS = 16
NW = NC * NS
L = 16
EMB = 5

N_ROWS, N_COLS = 16384, 200
I_SPAN = N_ROWS // NW
NJT = N_COLS // 8


def _sc_body(idt_hbm, w_hbm, out_hbm, id_v, out_v, w_v,
             in_sem0, in_sem1, out_sem0, out_sem1):
    wid = lax.axis_index("s") * NC + lax.axis_index("c")
    pltpu.sync_copy(w_hbm, w_v)
    p0 = [plsc.load_gather(w_v, [jnp.full((L,), k + 1, jnp.int32)])
          for k in range(EMB)]
    p1 = [plsc.load_gather(w_v, [jnp.full((L,), k + 1 + EMB, jnp.int32)])
          for k in range(EMB)]
    i0 = wid * I_SPAN

    in_sems = [in_sem0, in_sem1]
    out_sems = [out_sem0, out_sem1]
    pend_in = [None, None]
    pend_out = [[], []]

    def start_in(jt):
        b = jt & 1
        pend_in[b] = pltpu.async_copy(
            idt_hbm.at[pl.ds(jt * 8, 8), pl.ds(i0, I_SPAN)],
            id_v.at[b], in_sems[b])

    start_in(0)
    for jt in range(NJT):
        b = jt & 1
        if jt + 1 < NJT:
            start_in(jt + 1)
        pend_in[b].wait()
        for h in pend_out[b]:
            h.wait()
        pend_out[b] = []

        @plsc.parallel_loop(0, (8 * I_SPAN) // L, unroll=8)
        def c_body(t):
            j = t >> 5
            off = (t & 31) * L
            m = id_v[b, j, pl.ds(off, L)] == 0
            for k in range(EMB):
                out_v[b, k, j, pl.ds(off, L)] = jnp.where(m, p0[k], p1[k])

        for k in range(EMB):
            pend_out[b].append(pltpu.async_copy(
                out_v.at[b, k],
                out_hbm.at[k, pl.ds(jt * 8, 8), pl.ds(i0, I_SPAN)],
                out_sems[b]))

    for b in range(2):
        for h in pend_out[b]:
            h.wait()


_mesh = plsc.VectorSubcoreMesh(core_axis_name="c", subcore_axis_name="s")

_sc_lookup = functools.partial(
    pl.kernel,
    mesh=_mesh,
    out_type=jax.ShapeDtypeStruct((EMB, N_COLS, N_ROWS), jnp.float32),
    scratch_types=[
        pltpu.VMEM((2, 8, I_SPAN), jnp.int32),
        pltpu.VMEM((2, EMB, 8, I_SPAN), jnp.float32),
        pltpu.VMEM((L,), jnp.float32),
        pltpu.SemaphoreType.DMA,
        pltpu.SemaphoreType.DMA,
        pltpu.SemaphoreType.DMA,
        pltpu.SemaphoreType.DMA,
    ],
    compiler_params=pltpu.CompilerParams(needs_layout_passes=False),
)(_sc_body)


def kernel(id1, W):
    idt = id1.T
    wflat = jnp.pad(W.reshape(-1), (1, L - 2 * EMB - 1)).astype(jnp.float32)
    out_t = _sc_lookup(idt, wflat)
    return jnp.transpose(out_t, (2, 1, 0))


# speedup vs baseline: 1.0400x; 1.0400x over previous
import functools

import jax
import jax.numpy as jnp
from jax import lax
from jax.experimental import pallas as pl
from jax.experimental.pallas import tpu as pltpu
from jax.experimental.pallas import tpu_sc as plsc

NC = 2
NS = 16
NW = NC * NS
L = 16
EMB = 5

N_ROWS, N_COLS = 16384, 200
I_SPAN = N_ROWS // NW
NJT = N_COLS // 8


def _sc_body(idt_hbm, w_hbm, out_hbm, id_v, out_v, w_v,
             in_sem0, in_sem1, out_sem0, out_sem1):
    wid = lax.axis_index("s") * NC + lax.axis_index("c")
    pltpu.sync_copy(w_hbm, w_v)
    p0 = [plsc.load_gather(w_v, [jnp.full((L,), k + 1, jnp.int32)])
          for k in range(EMB)]
    p1 = [plsc.load_gather(w_v, [jnp.full((L,), k + 1 + EMB, jnp.int32)])
          for k in range(EMB)]
    i0 = wid * I_SPAN

    in_sems = [in_sem0, in_sem1]
    out_sems = [out_sem0, out_sem1]
    pend_in = [None, None]
    pend_out = [[], []]

    def start_in(jt):
        b = jt & 1
        pend_in[b] = pltpu.async_copy(
            idt_hbm.at[pl.ds(jt * 8, 8), pl.ds(i0, I_SPAN)],
            id_v.at[b], in_sems[b])

    start_in(0)
    for jt in range(NJT):
        b = jt & 1
        if jt + 1 < NJT:
            start_in(jt + 1)
        pend_in[b].wait()
        for h in pend_out[b]:
            h.wait()
        pend_out[b] = []

        @plsc.parallel_loop(0, (8 * I_SPAN) // L, unroll=4)
        def c_body(t):
            j = t >> 5
            off = (t & 31) * L
            m = id_v[b, j, pl.ds(off, L)] == 0
            for k in range(EMB):
                out_v[b, k, j, pl.ds(off, L)] = jnp.where(m, p0[k], p1[k])

        for k in range(EMB):
            pend_out[b].append(pltpu.async_copy(
                out_v.at[b, k],
                out_hbm.at[k, pl.ds(jt * 8, 8), pl.ds(i0, I_SPAN)],
                out_sems[b]))

    for b in range(2):
        for h in pend_out[b]:
            h.wait()


_mesh = plsc.VectorSubcoreMesh(core_axis_name="c", subcore_axis_name="s")

_sc_lookup = functools.partial(
    pl.kernel,
    mesh=_mesh,
    out_type=jax.ShapeDtypeStruct((EMB, N_COLS, N_ROWS), jnp.float32),
    scratch_types=[
        pltpu.VMEM((2, 8, I_SPAN), jnp.int32),
        pltpu.VMEM((2, EMB, 8, I_SPAN), jnp.float32),
        pltpu.VMEM((L,), jnp.float32),
        pltpu.SemaphoreType.DMA,
        pltpu.SemaphoreType.DMA,
        pltpu.SemaphoreType.DMA,
        pltpu.SemaphoreType.DMA,
    ],
    compiler_params=pltpu.CompilerParams(needs_layout_passes=False),
)(_sc_body)


def kernel(id1, W):
    idt = id1.T
    wflat = jnp.pad(W.reshape(-1), (1, L - 2 * EMB - 1)).astype(jnp.float32)
    out_t = _sc_lookup(idt, wflat)
    return jnp.transpose(out_t, (2, 1, 0))


# speedup vs baseline: 1.0414x; 1.0014x over previous
import functools

import jax
import jax.numpy as jnp
from jax import lax
from jax.experimental import pallas as pl
from jax.experimental.pallas import tpu as pltpu
from jax.experimental.pallas import tpu_sc as plsc

NC = 2
NS = 16
NW = NC * NS
L = 16
EMB = 5

N_ROWS, N_COLS = 16384, 200
I_SPAN = N_ROWS // NW
NJT = N_COLS // 8


def _sc_body(idt_hbm, w_hbm, out_hbm, id_v, out_v, w_v,
             in_sem0, in_sem1, out_sem0, out_sem1):
    wid = lax.axis_index("s") * NC + lax.axis_index("c")
    pltpu.sync_copy(w_hbm, w_v)
    p0 = [plsc.load_gather(w_v, [jnp.full((L,), k + 1, jnp.int32)])
          for k in range(EMB)]
    p1 = [plsc.load_gather(w_v, [jnp.full((L,), k + 1 + EMB, jnp.int32)])
          for k in range(EMB)]
    i0 = wid * I_SPAN

    in_sems = [in_sem0, in_sem1]
    out_sems = [out_sem0, out_sem1]
    pend_in = [None, None]
    pend_out = [[], []]

    def start_in(jt):
        b = jt & 1
        pend_in[b] = pltpu.async_copy(
            idt_hbm.at[pl.ds(jt * 8, 8), pl.ds(i0, I_SPAN)],
            id_v.at[b], in_sems[b])

    start_in(0)
    for jt in range(NJT):
        b = jt & 1
        if jt + 1 < NJT:
            start_in(jt + 1)
        pend_in[b].wait()
        for h in pend_out[b]:
            h.wait()
        pend_out[b] = []

        @plsc.parallel_loop(0, (8 * I_SPAN) // L, unroll=4)
        def c_body(t):
            j = t >> 5
            off = (t & 31) * L
            m = id_v[b, j, pl.ds(off, L)] == 0
            for k in range(EMB):
                out_v[b, k, j, pl.ds(off, L)] = jnp.where(m, p0[k], p1[k])

        for k in range(EMB):
            pend_out[b].append(pltpu.async_copy(
                out_v.at[b, k],
                out_hbm.at[k, pl.ds(jt * 8, 8), pl.ds(i0, I_SPAN)],
                out_sems[b]))

    for b in range(2):
        for h in pend_out[b]:
            h.wait()


_mesh = plsc.VectorSubcoreMesh(core_axis_name="c", subcore_axis_name="s")

_sc_lookup = functools.partial(
    pl.kernel,
    mesh=_mesh,
    out_type=jax.ShapeDtypeStruct((EMB, N_COLS, N_ROWS), jnp.float32),
    scratch_types=[
        pltpu.VMEM((2, 8, I_SPAN), jnp.int32),
        pltpu.VMEM((2, EMB, 8, I_SPAN), jnp.float32),
        pltpu.VMEM((L,), jnp.float32),
        pltpu.SemaphoreType.DMA,
        pltpu.SemaphoreType.DMA,
        pltpu.SemaphoreType.DMA,
        pltpu.SemaphoreType.DMA,
    ],
    compiler_params=pltpu.CompilerParams(needs_layout_passes=False, skip_device_barrier=True),
)(_sc_body)


def kernel(id1, W):
    idt = id1.T
    wflat = jnp.pad(W.reshape(-1), (1, L - 2 * EMB - 1)).astype(jnp.float32)
    out_t = _sc_lookup(idt, wflat)
    return jnp.transpose(out_t, (2, 1, 0))
